# Initial kernel scaffold; baseline (speedup 1.0000x reference)
import jax
import jax.numpy as jnp
from jax.experimental import pallas as pl


def kernel(x, edge_index, W1_rel, W1_root, b1, W2_rel, W2_root, b2, W3_rel, W3_root, b3):
    raise NotImplementedError("write your pallas kernel here")


# speedup vs baseline: 2.9778x; 2.9778x over previous
import functools

import jax
import jax.numpy as jnp
from jax import lax
from jax.experimental import pallas as pl
from jax.experimental.pallas import tpu as pltpu
from jax.experimental.pallas import tpu_sc as plsc

NC = 2
NS = 16
CHUNK = 128


def _make_seg_sum(n_nodes, d, n_chunks, acc_rows, zrows, rows_per_tile):
  mesh = plsc.VectorSubcoreMesh(core_axis_name="c", subcore_axis_name="s")

  @functools.partial(
      pl.kernel,
      mesh=mesh,
      out_type=jax.ShapeDtypeStruct((NC, n_nodes, d), jnp.float32),
      scratch_types=[
          pltpu.VMEM((8, CHUNK), jnp.int32),
          pltpu.VMEM((8, CHUNK), jnp.int32),
          pltpu.VMEM((CHUNK, d), jnp.float32),
          pltpu.VMEM((CHUNK, d), jnp.float32),
          pltpu.VMEM((16, d), jnp.float32),
          pltpu.VMEM_SHARED((acc_rows, d), jnp.float32),
          pltpu.SemaphoreType.DMA,
          pltpu.SemaphoreType.DMA,
      ],
  )
  def seg(g_hbm, src_hbm, dst_hbm, out_hbm,
          src_v, dst_v, bufa, bufb, zbuf, acc, sema, semb):
    c = lax.axis_index("c")
    s = lax.axis_index("s")
    wid = c * NS + s

    z16 = jnp.zeros((16,), jnp.float32)
    for r in range(16):
      for q in range(d // 16):
        zbuf[r, pl.ds(q * 16, 16)] = z16

    row0 = s * rows_per_tile

    def zero_body(k, carry):
      pltpu.sync_copy(zbuf, acc.at[pl.ds(row0 + k * 16, 16)])
      return carry

    lax.fori_loop(0, zrows // 16, zero_body, 0)
    rem0 = NS * rows_per_tile

    plsc.subcore_barrier()

    def body(i, carry):
      pltpu.sync_copy(src_hbm.at[wid, pl.ds(i * 8, 8)], src_v)
      pltpu.sync_copy(dst_hbm.at[wid, pl.ds(i * 8, 8)], dst_v)
      for k in range(0, 8, 2):
        ga = pltpu.async_copy(g_hbm.at[src_v.at[k]], bufa, sema)
        ga.wait()
        gb = pltpu.async_copy(g_hbm.at[src_v.at[k + 1]], bufb, semb)
        pltpu.sync_copy(bufa, acc.at[dst_v.at[k]], add=True)
        gb.wait()
        pltpu.sync_copy(bufb, acc.at[dst_v.at[k + 1]], add=True)
      return carry

    lax.fori_loop(0, n_chunks // 8, body, 0)
    plsc.subcore_barrier()

    pltpu.sync_copy(acc.at[pl.ds(row0, rows_per_tile)],
                    out_hbm.at[c].at[pl.ds(row0, rows_per_tile)])
    rem = n_nodes - NS * rows_per_tile
    if rem:
      @pl.when(s == 0)
      def _():
        pltpu.sync_copy(acc.at[pl.ds(rem0, rem)],
                        out_hbm.at[c].at[pl.ds(rem0, rem)])

  return seg


def _mm_body(x_ref, w_ref, o_ref):
  o_ref[...] = jnp.dot(x_ref[...], w_ref[...],
                       preferred_element_type=jnp.float32)


def _matmul(x, w, blk):
  n, d = x.shape
  return pl.pallas_call(
      _mm_body,
      grid=(n // blk,),
      in_specs=[
          pl.BlockSpec((blk, d), lambda i: (i, 0)),
          pl.BlockSpec((d, w.shape[1]), lambda i: (0, 0)),
      ],
      out_specs=pl.BlockSpec((blk, w.shape[1]), lambda i: (i, 0)),
      out_shape=jax.ShapeDtypeStruct((n, w.shape[1]), jnp.float32),
  )(x, w)


def _fused_body(relu, h_ref, p0_ref, p1_ref, wroot_ref, b_ref, wrel_ref,
                hn_ref, gn_ref):
  t = (p0_ref[...] + p1_ref[...] + b_ref[...]
       + jnp.dot(h_ref[...], wroot_ref[...],
                 preferred_element_type=jnp.float32))
  if relu:
    t = jnp.maximum(t, 0.0)
  hn_ref[...] = t
  gn_ref[...] = jnp.dot(t, wrel_ref[...], preferred_element_type=jnp.float32)


def _fused(h, p0, p1, w_root, b, w_rel_next, relu, blk):
  n, d = h.shape
  dn = w_root.shape[1]
  mat = lambda i: (i, 0)
  rep = lambda i: (0, 0)
  return pl.pallas_call(
      functools.partial(_fused_body, relu),
      grid=(n // blk,),
      in_specs=[
          pl.BlockSpec((blk, d), mat),
          pl.BlockSpec((blk, dn), mat),
          pl.BlockSpec((blk, dn), mat),
          pl.BlockSpec((d, dn), rep),
          pl.BlockSpec((1, dn), rep),
          pl.BlockSpec((dn, w_rel_next.shape[1]), rep),
      ],
      out_specs=[
          pl.BlockSpec((blk, dn), mat),
          pl.BlockSpec((blk, w_rel_next.shape[1]), mat),
      ],
      out_shape=[
          jax.ShapeDtypeStruct((n, dn), jnp.float32),
          jax.ShapeDtypeStruct((n, w_rel_next.shape[1]), jnp.float32),
      ],
  )(h, p0, p1, w_root, b.reshape(1, -1), w_rel_next)


def _final_body(h_ref, p0_ref, p1_ref, wroot_ref, b_ref, o_ref):
  o_ref[...] = (p0_ref[...] + p1_ref[...] + b_ref[...]
                + jnp.dot(h_ref[...], wroot_ref[...],
                          preferred_element_type=jnp.float32))


def _final(h, p0, p1, w_root, b, blk):
  n, d = h.shape
  dn = w_root.shape[1]
  mat = lambda i: (i, 0)
  rep = lambda i: (0, 0)
  return pl.pallas_call(
      _final_body,
      grid=(n // blk,),
      in_specs=[
          pl.BlockSpec((blk, d), mat),
          pl.BlockSpec((blk, dn), mat),
          pl.BlockSpec((blk, dn), mat),
          pl.BlockSpec((d, dn), rep),
          pl.BlockSpec((1, dn), rep),
      ],
      out_specs=pl.BlockSpec((blk, dn), mat),
      out_shape=jax.ShapeDtypeStruct((n, dn), jnp.float32),
  )(h, p0, p1, w_root, b.reshape(1, -1))


def kernel(x, edge_index, W1_rel, W1_root, b1, W2_rel, W2_root, b2,
           W3_rel, W3_root, b3):
  n, d = x.shape
  e = edge_index.shape[1]
  n_tiles = NC * NS

  per_tile_chunks = -(-e // (n_tiles * CHUNK))
  per_tile_chunks = -(-per_tile_chunks // 8) * 8
  e_pad = n_tiles * per_tile_chunks * CHUNK
  pad = e_pad - e

  src = edge_index[0].astype(jnp.int32)
  dst = edge_index[1].astype(jnp.int32)
  src_p = jnp.concatenate([src, jnp.zeros((pad,), jnp.int32)])
  dst_p = jnp.concatenate(
      [dst, n + (jnp.arange(pad, dtype=jnp.int32) % 8)])
  src_p = src_p.reshape(n_tiles, per_tile_chunks, CHUNK)
  dst_p = dst_p.reshape(n_tiles, per_tile_chunks, CHUNK)

  rows_per_tile = (n // NS) // 16 * 16
  zrows = -(-(n + 8 - (NS - 1) * rows_per_tile) // 16) * 16
  acc_rows = (NS - 1) * rows_per_tile + zrows
  seg = _make_seg_sum(n, d, per_tile_chunks, acc_rows, zrows, rows_per_tile)

  blk = 1000
  g1 = _matmul(x, W1_rel, blk)
  q1 = seg(g1, src_p, dst_p)
  h1, g2 = _fused(x, q1[0], q1[1], W1_root, b1, W2_rel, True, blk)
  q2 = seg(g2, src_p, dst_p)
  h2, g3 = _fused(h1, q2[0], q2[1], W2_root, b2, W3_rel, True, blk)
  q3 = seg(g3, src_p, dst_p)
  return _final(h2, q3[0], q3[1], W3_root, b3, blk)


# speedup vs baseline: 8.4189x; 2.8272x over previous
import functools

import jax
import jax.numpy as jnp
from jax import lax
from jax.experimental import pallas as pl
from jax.experimental.pallas import tpu as pltpu
from jax.experimental.pallas import tpu_sc as plsc

NC = 2
NS = 16
CHUNK = 128


def _make_seg_sum(n_nodes, d, n_chunks, acc_rows, zrows, rows_per_tile):
  mesh = plsc.VectorSubcoreMesh(core_axis_name="c", subcore_axis_name="s")

  @functools.partial(
      pl.kernel,
      mesh=mesh,
      out_type=jax.ShapeDtypeStruct((NC, n_nodes, d), jnp.float32),
      scratch_types=[
          pltpu.VMEM((8, CHUNK), jnp.int32),
          pltpu.VMEM((8, CHUNK), jnp.int32),
          pltpu.VMEM((CHUNK, d), jnp.float32),
          pltpu.VMEM((CHUNK, d), jnp.float32),
          pltpu.VMEM((16, d), jnp.float32),
          pltpu.VMEM_SHARED((acc_rows, d), jnp.float32),
          pltpu.SemaphoreType.DMA,
          pltpu.SemaphoreType.DMA,
      ],
  )
  def seg(g_hbm, src_hbm, dst_hbm, out_hbm,
          src_v, dst_v, bufa, bufb, zbuf, acc, sema, semb):
    c = lax.axis_index("c")
    s = lax.axis_index("s")
    wid = c * NS + s

    z16 = jnp.zeros((16,), jnp.float32)
    for r in range(16):
      for q in range(d // 16):
        zbuf[r, pl.ds(q * 16, 16)] = z16

    row0 = s * rows_per_tile

    def zero_body(k, carry):
      pltpu.sync_copy(zbuf, acc.at[pl.ds(row0 + k * 16, 16)])
      return carry

    lax.fori_loop(0, zrows // 16, zero_body, 0)
    rem0 = NS * rows_per_tile

    plsc.subcore_barrier()

    def body(i, carry):
      pltpu.sync_copy(src_hbm.at[wid, pl.ds(i * 8, 8)], src_v)
      pltpu.sync_copy(dst_hbm.at[wid, pl.ds(i * 8, 8)], dst_v)
      for k in range(0, 8, 2):
        ga = pltpu.async_copy(g_hbm.at[src_v.at[k]], bufa, sema)
        ga.wait()
        gb = pltpu.async_copy(g_hbm.at[src_v.at[k + 1]], bufb, semb)
        pltpu.sync_copy(bufa, acc.at[dst_v.at[k]], add=True)
        gb.wait()
        pltpu.sync_copy(bufb, acc.at[dst_v.at[k + 1]], add=True)
      return carry

    lax.fori_loop(0, n_chunks // 8, body, 0)
    plsc.subcore_barrier()

    pltpu.sync_copy(acc.at[pl.ds(row0, rows_per_tile)],
                    out_hbm.at[c].at[pl.ds(row0, rows_per_tile)])
    rem = n_nodes - NS * rows_per_tile
    if rem:
      @pl.when(s == 0)
      def _():
        pltpu.sync_copy(acc.at[pl.ds(rem0, rem)],
                        out_hbm.at[c].at[pl.ds(rem0, rem)])

  return seg


def _mm_body(x_ref, w_ref, o_ref):
  o_ref[...] = jnp.dot(x_ref[...], w_ref[...],
                       preferred_element_type=jnp.float32)


def _matmul(x, w, blk):
  n, d = x.shape
  return pl.pallas_call(
      _mm_body,
      grid=(n // blk,),
      in_specs=[
          pl.BlockSpec((blk, d), lambda i: (i, 0)),
          pl.BlockSpec((d, w.shape[1]), lambda i: (0, 0)),
      ],
      out_specs=pl.BlockSpec((blk, w.shape[1]), lambda i: (i, 0)),
      out_shape=jax.ShapeDtypeStruct((n, w.shape[1]), jnp.float32),
  )(x, w)


def _fused_body(relu, h_ref, p0_ref, p1_ref, wroot_ref, b_ref, wrel_ref,
                hn_ref, gn_ref):
  t = (p0_ref[...] + p1_ref[...] + b_ref[...]
       + jnp.dot(h_ref[...], wroot_ref[...],
                 preferred_element_type=jnp.float32))
  if relu:
    t = jnp.maximum(t, 0.0)
  hn_ref[...] = t
  gn_ref[...] = jnp.dot(t, wrel_ref[...], preferred_element_type=jnp.float32)


def _fused(h, p0, p1, w_root, b, w_rel_next, relu, blk):
  n, d = h.shape
  dn = w_root.shape[1]
  mat = lambda i: (i, 0)
  rep = lambda i: (0, 0)
  return pl.pallas_call(
      functools.partial(_fused_body, relu),
      grid=(n // blk,),
      in_specs=[
          pl.BlockSpec((blk, d), mat),
          pl.BlockSpec((blk, dn), mat),
          pl.BlockSpec((blk, dn), mat),
          pl.BlockSpec((d, dn), rep),
          pl.BlockSpec((1, dn), rep),
          pl.BlockSpec((dn, w_rel_next.shape[1]), rep),
      ],
      out_specs=[
          pl.BlockSpec((blk, dn), mat),
          pl.BlockSpec((blk, w_rel_next.shape[1]), mat),
      ],
      out_shape=[
          jax.ShapeDtypeStruct((n, dn), jnp.float32),
          jax.ShapeDtypeStruct((n, w_rel_next.shape[1]), jnp.float32),
      ],
  )(h, p0, p1, w_root, b.reshape(1, -1), w_rel_next)


def _final_body(h_ref, p0_ref, p1_ref, wroot_ref, b_ref, o_ref):
  o_ref[...] = (p0_ref[...] + p1_ref[...] + b_ref[...]
                + jnp.dot(h_ref[...], wroot_ref[...],
                          preferred_element_type=jnp.float32))


def _final(h, p0, p1, w_root, b, blk):
  n, d = h.shape
  dn = w_root.shape[1]
  mat = lambda i: (i, 0)
  rep = lambda i: (0, 0)
  return pl.pallas_call(
      _final_body,
      grid=(n // blk,),
      in_specs=[
          pl.BlockSpec((blk, d), mat),
          pl.BlockSpec((blk, dn), mat),
          pl.BlockSpec((blk, dn), mat),
          pl.BlockSpec((d, dn), rep),
          pl.BlockSpec((1, dn), rep),
      ],
      out_specs=pl.BlockSpec((blk, dn), mat),
      out_shape=jax.ShapeDtypeStruct((n, dn), jnp.float32),
  )(h, p0, p1, w_root, b.reshape(1, -1))


def kernel(x, edge_index, W1_rel, W1_root, b1, W2_rel, W2_root, b2,
           W3_rel, W3_root, b3):
  n, d = x.shape
  e = edge_index.shape[1]
  n_tiles = NC * NS

  per_tile_chunks = -(-e // (n_tiles * CHUNK))
  per_tile_chunks = -(-per_tile_chunks // 8) * 8
  e_pad = n_tiles * per_tile_chunks * CHUNK
  pad = e_pad - e

  src = edge_index[0].astype(jnp.int32)
  dst = edge_index[1].astype(jnp.int32)
  pad_i = jnp.arange(pad, dtype=jnp.int32)
  src_p = jnp.concatenate([src, n + pad_i % 16])
  dst_p = jnp.concatenate([dst, pad_i * 37 % n])
  src_p = src_p.reshape(n_tiles, per_tile_chunks, CHUNK)
  dst_p = dst_p.reshape(n_tiles, per_tile_chunks, CHUNK)
  zrow = jnp.zeros((16, d), jnp.float32)

  rows_per_tile = (n // NS) // 16 * 16
  zrows = -(-(n + 8 - (NS - 1) * rows_per_tile) // 16) * 16
  acc_rows = (NS - 1) * rows_per_tile + zrows
  seg = _make_seg_sum(n, d, per_tile_chunks, acc_rows, zrows, rows_per_tile)

  blk = 1000
  g1 = _matmul(x, W1_rel, blk)
  q1 = seg(jnp.concatenate([g1, zrow]), src_p, dst_p)
  h1, g2 = _fused(x, q1[0], q1[1], W1_root, b1, W2_rel, True, blk)
  q2 = seg(jnp.concatenate([g2, zrow]), src_p, dst_p)
  h2, g3 = _fused(h1, q2[0], q2[1], W2_root, b2, W3_rel, True, blk)
  q3 = seg(jnp.concatenate([g3, zrow]), src_p, dst_p)
  return _final(h2, q3[0], q3[1], W3_root, b3, blk)


# speedup vs baseline: 8.9239x; 1.0600x over previous
import functools

import jax
import jax.numpy as jnp
from jax import lax
from jax.experimental import pallas as pl
from jax.experimental.pallas import tpu as pltpu
from jax.experimental.pallas import tpu_sc as plsc

NC = 2
NS = 16
CHUNK = 128


def _make_seg_sum(n_nodes, d, n_chunks, acc_rows, zrows, rows_per_tile):
  mesh = plsc.VectorSubcoreMesh(core_axis_name="c", subcore_axis_name="s")

  @functools.partial(
      pl.kernel,
      mesh=mesh,
      out_type=jax.ShapeDtypeStruct((NC, n_nodes, d), jnp.float32),
      scratch_types=[
          pltpu.VMEM((2, 8, CHUNK), jnp.int32),
          pltpu.VMEM((2, 8, CHUNK), jnp.int32),
          pltpu.VMEM((CHUNK, d), jnp.float32),
          pltpu.VMEM((CHUNK, d), jnp.float32),
          pltpu.VMEM((16, d), jnp.float32),
          pltpu.VMEM_SHARED((acc_rows, d), jnp.float32),
          pltpu.SemaphoreType.DMA,
          pltpu.SemaphoreType.DMA,
          pltpu.SemaphoreType.DMA,
          pltpu.SemaphoreType.DMA,
      ],
  )
  def seg(g_hbm, src_hbm, dst_hbm, out_hbm,
          src_v, dst_v, bufa, bufb, zbuf, acc, gsa, gsb, ssa, ssb):
    c = lax.axis_index("c")
    s = lax.axis_index("s")
    wid = c * NS + s

    z16 = jnp.zeros((16,), jnp.float32)
    for r in range(16):
      for q in range(d // 16):
        zbuf[r, pl.ds(q * 16, 16)] = z16

    row0 = s * rows_per_tile

    def zero_body(k, carry):
      pltpu.sync_copy(zbuf, acc.at[pl.ds(row0 + k * 16, 16)])
      return carry

    lax.fori_loop(0, zrows // 16, zero_body, 0)
    rem0 = NS * rows_per_tile

    plsc.subcore_barrier()

    def body(i, carry):
      grp = i // 4
      p = grp % 2
      r0 = (2 * i) % 8
      r1 = r0 + 1

      @pl.when(i % 4 == 0)
      def _():
        pltpu.sync_copy(src_hbm.at[wid, pl.ds(grp * 8, 8)], src_v.at[p])
        pltpu.sync_copy(dst_hbm.at[wid, pl.ds(grp * 8, 8)], dst_v.at[p])

      @pl.when(i > 0)
      def _():
        pltpu.make_async_copy(bufa, acc.at[dst_v.at[p, r0]], ssa).wait()
        pltpu.make_async_copy(bufb, acc.at[dst_v.at[p, r1]], ssb).wait()

      ga = pltpu.async_copy(g_hbm.at[src_v.at[p, r0]], bufa, gsa)
      gb = pltpu.async_copy(g_hbm.at[src_v.at[p, r1]], bufb, gsb)
      ga.wait()
      pltpu.async_copy(bufa, acc.at[dst_v.at[p, r0]], ssa, add=True)
      gb.wait()
      pltpu.async_copy(bufb, acc.at[dst_v.at[p, r1]], ssb, add=True)
      return carry

    n_pairs = n_chunks // 2
    lax.fori_loop(0, n_pairs, body, 0)
    pltpu.make_async_copy(bufa, acc.at[dst_v.at[0, 0]], ssa).wait()
    pltpu.make_async_copy(bufb, acc.at[dst_v.at[0, 1]], ssb).wait()
    plsc.subcore_barrier()

    pltpu.sync_copy(acc.at[pl.ds(row0, rows_per_tile)],
                    out_hbm.at[c].at[pl.ds(row0, rows_per_tile)])
    rem = n_nodes - NS * rows_per_tile
    if rem:
      @pl.when(s == 0)
      def _():
        pltpu.sync_copy(acc.at[pl.ds(rem0, rem)],
                        out_hbm.at[c].at[pl.ds(rem0, rem)])

  return seg


def _mm_body(x_ref, w_ref, o_ref):
  o_ref[...] = jnp.dot(x_ref[...], w_ref[...],
                       preferred_element_type=jnp.float32)


def _matmul(x, w, blk):
  n, d = x.shape
  return pl.pallas_call(
      _mm_body,
      grid=(n // blk,),
      in_specs=[
          pl.BlockSpec((blk, d), lambda i: (i, 0)),
          pl.BlockSpec((d, w.shape[1]), lambda i: (0, 0)),
      ],
      out_specs=pl.BlockSpec((blk, w.shape[1]), lambda i: (i, 0)),
      out_shape=jax.ShapeDtypeStruct((n, w.shape[1]), jnp.float32),
  )(x, w)


def _fused_body(relu, h_ref, p0_ref, p1_ref, wroot_ref, b_ref, wrel_ref,
                hn_ref, gn_ref):
  t = (p0_ref[...] + p1_ref[...] + b_ref[...]
       + jnp.dot(h_ref[...], wroot_ref[...],
                 preferred_element_type=jnp.float32))
  if relu:
    t = jnp.maximum(t, 0.0)
  hn_ref[...] = t
  gn_ref[...] = jnp.dot(t, wrel_ref[...], preferred_element_type=jnp.float32)


def _fused(h, p0, p1, w_root, b, w_rel_next, relu, blk):
  n, d = h.shape
  dn = w_root.shape[1]
  mat = lambda i: (i, 0)
  rep = lambda i: (0, 0)
  return pl.pallas_call(
      functools.partial(_fused_body, relu),
      grid=(n // blk,),
      in_specs=[
          pl.BlockSpec((blk, d), mat),
          pl.BlockSpec((blk, dn), mat),
          pl.BlockSpec((blk, dn), mat),
          pl.BlockSpec((d, dn), rep),
          pl.BlockSpec((1, dn), rep),
          pl.BlockSpec((dn, w_rel_next.shape[1]), rep),
      ],
      out_specs=[
          pl.BlockSpec((blk, dn), mat),
          pl.BlockSpec((blk, w_rel_next.shape[1]), mat),
      ],
      out_shape=[
          jax.ShapeDtypeStruct((n, dn), jnp.float32),
          jax.ShapeDtypeStruct((n, w_rel_next.shape[1]), jnp.float32),
      ],
  )(h, p0, p1, w_root, b.reshape(1, -1), w_rel_next)


def _final_body(h_ref, p0_ref, p1_ref, wroot_ref, b_ref, o_ref):
  o_ref[...] = (p0_ref[...] + p1_ref[...] + b_ref[...]
                + jnp.dot(h_ref[...], wroot_ref[...],
                          preferred_element_type=jnp.float32))


def _final(h, p0, p1, w_root, b, blk):
  n, d = h.shape
  dn = w_root.shape[1]
  mat = lambda i: (i, 0)
  rep = lambda i: (0, 0)
  return pl.pallas_call(
      _final_body,
      grid=(n // blk,),
      in_specs=[
          pl.BlockSpec((blk, d), mat),
          pl.BlockSpec((blk, dn), mat),
          pl.BlockSpec((blk, dn), mat),
          pl.BlockSpec((d, dn), rep),
          pl.BlockSpec((1, dn), rep),
      ],
      out_specs=pl.BlockSpec((blk, dn), mat),
      out_shape=jax.ShapeDtypeStruct((n, dn), jnp.float32),
  )(h, p0, p1, w_root, b.reshape(1, -1))


def kernel(x, edge_index, W1_rel, W1_root, b1, W2_rel, W2_root, b2,
           W3_rel, W3_root, b3):
  n, d = x.shape
  e = edge_index.shape[1]
  n_tiles = NC * NS

  per_tile_chunks = -(-e // (n_tiles * CHUNK))
  per_tile_chunks = -(-per_tile_chunks // 8) * 8
  e_pad = n_tiles * per_tile_chunks * CHUNK
  pad = e_pad - e

  src = edge_index[0].astype(jnp.int32)
  dst = edge_index[1].astype(jnp.int32)
  pad_i = jnp.arange(pad, dtype=jnp.int32)
  src_p = jnp.concatenate([src, n + pad_i % 16])
  dst_p = jnp.concatenate([dst, pad_i * 37 % n])
  src_p = src_p.reshape(n_tiles, per_tile_chunks, CHUNK)
  dst_p = dst_p.reshape(n_tiles, per_tile_chunks, CHUNK)
  zrow = jnp.zeros((16, d), jnp.float32)

  rows_per_tile = (n // NS) // 16 * 16
  zrows = -(-(n + 8 - (NS - 1) * rows_per_tile) // 16) * 16
  acc_rows = (NS - 1) * rows_per_tile + zrows
  seg = _make_seg_sum(n, d, per_tile_chunks, acc_rows, zrows, rows_per_tile)

  blk = 1000
  g1 = _matmul(x, W1_rel, blk)
  q1 = seg(jnp.concatenate([g1, zrow]), src_p, dst_p)
  h1, g2 = _fused(x, q1[0], q1[1], W1_root, b1, W2_rel, True, blk)
  q2 = seg(jnp.concatenate([g2, zrow]), src_p, dst_p)
  h2, g3 = _fused(h1, q2[0], q2[1], W2_root, b2, W3_rel, True, blk)
  q3 = seg(jnp.concatenate([g3, zrow]), src_p, dst_p)
  return _final(h2, q3[0], q3[1], W3_root, b3, blk)
